# baseline (speedup 1.0000x reference)
import jax
import jax.numpy as jnp
from jax import lax
from jax.experimental import pallas as pl
from jax.experimental.pallas import tpu as pltpu
from jax.experimental.pallas import tpu_sc as plsc

NC = 2
NS = 16
NW = NC * NS
CH = 128
DW = 8
DH = 64


def _sc_segment_sum(fs_a, fs_b, src_t, dst_t, zrow, zdeg, ones,
                    acc_rows, k_chunks):
    share = acc_rows // NS

    def body(fsa_hbm, fsb_hbm, src_hbm, dst_hbm, zrow_hbm, zdeg_hbm, ones_hbm,
             sums_hbm, degs_hbm,
             src_v, dst_v, rows_a, rows_b, zrow_v, zdeg_v, ones_v, acc, deg,
             gsem_a, gsem_b, dsem):
        c = lax.axis_index("c")
        s = lax.axis_index("s")
        wid = c * NS + s

        pltpu.sync_copy(src_hbm.at[wid], src_v)
        pltpu.sync_copy(dst_hbm.at[wid], dst_v)
        pltpu.sync_copy(zrow_hbm, zrow_v)
        pltpu.sync_copy(zdeg_hbm, zdeg_v)
        pltpu.sync_copy(ones_hbm, ones_v)

        base = s * share

        def zero_share(with_deg):
            for k in range(share // CH):
                pltpu.sync_copy(zrow_v, acc.at[pl.ds(base + k * CH, CH)])
                if with_deg:
                    pltpu.sync_copy(zdeg_v, deg.at[pl.ds(base + k * CH, CH)])

        def accumulate(feat_hbm, with_deg):
            def chunk(j, carry):
                pltpu.async_copy(feat_hbm.at[src_v.at[j]], rows_a, gsem_a).wait()
                pltpu.sync_copy(rows_a, acc.at[dst_v.at[j]], add=True)
                if with_deg:
                    pltpu.sync_copy(ones_v, deg.at[dst_v.at[j]], add=True)
                return carry

            lax.fori_loop(0, k_chunks, chunk, 0)

        def write_share(half, with_deg):
            for k in range(share // CH):
                r = base + k * CH
                pltpu.sync_copy(acc.at[pl.ds(r, CH)], rows_a)
                pltpu.sync_copy(rows_a, sums_hbm.at[c, half, pl.ds(r, CH)])
                if with_deg:
                    pltpu.sync_copy(deg.at[pl.ds(r, CH)], ones_v)
                    pltpu.sync_copy(ones_v, degs_hbm.at[c, pl.ds(r, CH)])

        zero_share(with_deg=True)
        plsc.subcore_barrier()
        accumulate(fsa_hbm, with_deg=True)
        plsc.subcore_barrier()
        write_share(0, with_deg=True)
        zero_share(with_deg=False)
        plsc.subcore_barrier()
        accumulate(fsb_hbm, with_deg=False)
        plsc.subcore_barrier()
        write_share(1, with_deg=False)

    mesh = plsc.VectorSubcoreMesh(core_axis_name="c", subcore_axis_name="s",
                                  num_cores=NC, num_subcores=NS)
    fn = pl.kernel(
        body,
        out_type=(jax.ShapeDtypeStruct((NC, 2, acc_rows, DH), jnp.float32),
                  jax.ShapeDtypeStruct((NC, acc_rows, DW), jnp.float32)),
        mesh=mesh,
        compiler_params=pltpu.CompilerParams(use_tc_tiling_on_sc=False),
        scratch_types=[
            pltpu.VMEM((k_chunks, CH), jnp.int32),
            pltpu.VMEM((k_chunks, CH), jnp.int32),
            pltpu.VMEM((CH, DH), jnp.float32),
            pltpu.VMEM((CH, DH), jnp.float32),
            pltpu.VMEM((CH, DH), jnp.float32),
            pltpu.VMEM((CH, DW), jnp.float32),
            pltpu.VMEM((CH, DW), jnp.float32),
            pltpu.VMEM_SHARED((acc_rows, DH), jnp.float32),
            pltpu.VMEM_SHARED((acc_rows, DW), jnp.float32),
            pltpu.SemaphoreType.DMA,
            pltpu.SemaphoreType.DMA,
            pltpu.SemaphoreType.DMA,
        ],
    )
    return fn(fs_a, fs_b, src_t, dst_t, zrow, zdeg, ones)


def _tc_combine(sums, degs, fd, wst, wnt_a, wnt_b, bs, bn, rows, blk):
    d = fd.shape[1]

    def body(p_ref, d_ref, fd_ref, wst_ref, wa_ref, wb_ref, bs_ref, bn_ref,
             o_ref):
        sa = p_ref[0, 0] + p_ref[1, 0]
        sb = p_ref[0, 1] + p_ref[1, 1]
        deg = d_ref[0, :, 0:1] + d_ref[1, :, 0:1]
        inv = 1.0 / jnp.maximum(deg, 1.0)
        o_ref[...] = (
            jnp.dot(fd_ref[...], wst_ref[...], preferred_element_type=jnp.float32)
            + jnp.dot(sa * inv, wa_ref[...], preferred_element_type=jnp.float32)
            + jnp.dot(sb * inv, wb_ref[...], preferred_element_type=jnp.float32)
            + bs_ref[...] + bn_ref[...]
        )

    return pl.pallas_call(
        body,
        grid=(rows // blk,),
        in_specs=[
            pl.BlockSpec((2, 2, blk, DH), lambda i: (0, 0, i, 0)),
            pl.BlockSpec((2, blk, DW), lambda i: (0, i, 0)),
            pl.BlockSpec((blk, d), lambda i: (i, 0)),
            pl.BlockSpec((d, d), lambda i: (0, 0)),
            pl.BlockSpec((DH, d), lambda i: (0, 0)),
            pl.BlockSpec((DH, d), lambda i: (0, 0)),
            pl.BlockSpec((1, d), lambda i: (0, 0)),
            pl.BlockSpec((1, d), lambda i: (0, 0)),
        ],
        out_specs=pl.BlockSpec((blk, d), lambda i: (i, 0)),
        out_shape=jax.ShapeDtypeStruct((rows, d), jnp.float32),
    )(sums, degs, fd, wst, wnt_a, wnt_b, bs, bn)


def kernel(feat_src, feat_dst, edge_index, W_self, b_self, W_neigh, b_neigh):
    n_src, d = feat_src.shape
    n_dst = feat_dst.shape[0]
    e = edge_index.shape[1]

    k_chunks = -(-e // (NW * CH))
    k_chunks += k_chunks % 2
    e_pad = NW * k_chunks * CH
    acc_rows = -(-(n_dst + 1) // (NS * CH)) * (NS * CH)

    src = edge_index[0].astype(jnp.int32)
    dst = edge_index[1].astype(jnp.int32)
    pad = e_pad - e
    src_t = jnp.concatenate([src, jnp.zeros((pad,), jnp.int32)]).reshape(NW, k_chunks, CH)
    dst_t = jnp.concatenate([dst, jnp.full((pad,), n_dst, jnp.int32)]).reshape(NW, k_chunks, CH)

    fs_a = feat_src[:, :DH]
    fs_b = feat_src[:, DH:]
    zrow = jnp.zeros((CH, DH), jnp.float32)
    zdeg = jnp.zeros((CH, DW), jnp.float32)
    ones = jnp.ones((CH, DW), jnp.float32)

    sums, degs = _sc_segment_sum(fs_a, fs_b, src_t, dst_t, zrow, zdeg, ones,
                                 acc_rows, k_chunks)

    fd_pad = jnp.pad(feat_dst, ((0, acc_rows - n_dst), (0, 0)))
    wnt = W_neigh.T
    out = _tc_combine(sums, degs, fd_pad, W_self.T, wnt[:DH], wnt[DH:],
                      b_self.reshape(1, d), b_neigh.reshape(1, d),
                      acc_rows, 512)
    return out[:n_dst]


# speedup vs baseline: 2.0354x; 2.0354x over previous
import jax
import jax.numpy as jnp
from jax import lax
from jax.experimental import pallas as pl
from jax.experimental.pallas import tpu as pltpu
from jax.experimental.pallas import tpu_sc as plsc

NC = 2
NS = 16
NW = NC * NS
CH = 128
DW = 8
DH = 64


def _sc_segment_sum(fs_a, fs_b, src_t, dst_t, zrow, zdeg, ones,
                    acc_rows, k_chunks):
    share = acc_rows // NS

    def body(fsa_hbm, fsb_hbm, src_hbm, dst_hbm, zrow_hbm, zdeg_hbm, ones_hbm,
             sums_hbm, degs_hbm,
             src_v, dst_v, rows_a, zrow_v, zdeg_v, ones_v, acc, deg, gsem_a):
        c = lax.axis_index("c")
        s = lax.axis_index("s")
        wid = c * NS + s

        pltpu.sync_copy(src_hbm.at[wid], src_v)
        pltpu.sync_copy(dst_hbm.at[wid], dst_v)
        pltpu.sync_copy(zrow_hbm, zrow_v)
        pltpu.sync_copy(zdeg_hbm, zdeg_v)
        pltpu.sync_copy(ones_hbm, ones_v)

        base = s * share

        def zero_share(with_deg):
            for k in range(share // CH):
                pltpu.sync_copy(zrow_v, acc.at[pl.ds(base + k * CH, CH)])
                if with_deg:
                    pltpu.sync_copy(zdeg_v, deg.at[pl.ds(base + k * CH, CH)])

        def accumulate(feat_hbm, with_deg):
            def chunk(j, carry):
                pltpu.async_copy(feat_hbm.at[src_v.at[j]], rows_a, gsem_a).wait()
                pltpu.sync_copy(rows_a, acc.at[dst_v.at[j]], add=True)
                if with_deg:
                    pltpu.sync_copy(ones_v, deg.at[dst_v.at[j]], add=True)
                return carry

            lax.fori_loop(0, k_chunks, chunk, 0)

        def write_share(half, with_deg):
            for k in range(share // CH):
                r = base + k * CH
                pltpu.sync_copy(acc.at[pl.ds(r, CH)], rows_a)
                pltpu.sync_copy(rows_a, sums_hbm.at[c, half, pl.ds(r, CH)])
                if with_deg:
                    pltpu.sync_copy(deg.at[pl.ds(r, CH)], ones_v)
                    pltpu.sync_copy(ones_v, degs_hbm.at[c, pl.ds(r, CH)])

        zero_share(with_deg=True)
        plsc.subcore_barrier()
        accumulate(fsa_hbm, with_deg=True)
        plsc.subcore_barrier()
        write_share(0, with_deg=True)
        zero_share(with_deg=False)
        plsc.subcore_barrier()
        accumulate(fsb_hbm, with_deg=False)
        plsc.subcore_barrier()
        write_share(1, with_deg=False)

    mesh = plsc.VectorSubcoreMesh(core_axis_name="c", subcore_axis_name="s",
                                  num_cores=NC, num_subcores=NS)
    fn = pl.kernel(
        body,
        out_type=(jax.ShapeDtypeStruct((NC, 2, acc_rows, DH), jnp.float32),
                  jax.ShapeDtypeStruct((NC, acc_rows, DW), jnp.float32)),
        mesh=mesh,
        compiler_params=pltpu.CompilerParams(use_tc_tiling_on_sc=False),
        scratch_types=[
            pltpu.VMEM((k_chunks, CH), jnp.int32),
            pltpu.VMEM((k_chunks, CH), jnp.int32),
            pltpu.VMEM((CH, DH), jnp.float32),
            pltpu.VMEM((CH, DH), jnp.float32),
            pltpu.VMEM((CH, DW), jnp.float32),
            pltpu.VMEM((CH, DW), jnp.float32),
            pltpu.VMEM_SHARED((acc_rows, DH), jnp.float32),
            pltpu.VMEM_SHARED((acc_rows, DW), jnp.float32),
            pltpu.SemaphoreType.DMA,
        ],
    )
    return fn(fs_a, fs_b, src_t, dst_t, zrow, zdeg, ones)


def _tc_combine(sums, degs, fd, wst, wnt_a, wnt_b, bs, bn, rows, blk):
    d = fd.shape[1]

    def body(p_ref, d_ref, fd_ref, wst_ref, wa_ref, wb_ref, bs_ref, bn_ref,
             o_ref):
        sa = p_ref[0, 0] + p_ref[1, 0]
        sb = p_ref[0, 1] + p_ref[1, 1]
        deg = d_ref[0, :, 0:1] + d_ref[1, :, 0:1]
        inv = 1.0 / jnp.maximum(deg, 1.0)
        o_ref[...] = (
            jnp.dot(fd_ref[...], wst_ref[...], preferred_element_type=jnp.float32)
            + jnp.dot(sa * inv, wa_ref[...], preferred_element_type=jnp.float32)
            + jnp.dot(sb * inv, wb_ref[...], preferred_element_type=jnp.float32)
            + bs_ref[...] + bn_ref[...]
        )

    return pl.pallas_call(
        body,
        grid=(rows // blk,),
        in_specs=[
            pl.BlockSpec((2, 2, blk, DH), lambda i: (0, 0, i, 0)),
            pl.BlockSpec((2, blk, DW), lambda i: (0, i, 0)),
            pl.BlockSpec((blk, d), lambda i: (i, 0)),
            pl.BlockSpec((d, d), lambda i: (0, 0)),
            pl.BlockSpec((DH, d), lambda i: (0, 0)),
            pl.BlockSpec((DH, d), lambda i: (0, 0)),
            pl.BlockSpec((1, d), lambda i: (0, 0)),
            pl.BlockSpec((1, d), lambda i: (0, 0)),
        ],
        out_specs=pl.BlockSpec((blk, d), lambda i: (i, 0)),
        out_shape=jax.ShapeDtypeStruct((rows, d), jnp.float32),
    )(sums, degs, fd, wst, wnt_a, wnt_b, bs, bn)


def kernel(feat_src, feat_dst, edge_index, W_self, b_self, W_neigh, b_neigh):
    n_src, d = feat_src.shape
    n_dst = feat_dst.shape[0]
    e = edge_index.shape[1]

    k_chunks = -(-e // (NW * CH))
    e_pad = NW * k_chunks * CH
    acc_rows = -(-(n_dst + 1) // (NS * CH)) * (NS * CH)

    src = edge_index[0].astype(jnp.int32)
    dst = edge_index[1].astype(jnp.int32)
    pad = e_pad - e
    pad_cycle = jnp.arange(pad, dtype=jnp.int32) % CH
    src_t = jnp.concatenate([src, pad_cycle]).reshape(NW, k_chunks, CH)
    dst_t = jnp.concatenate([dst, n_dst + pad_cycle]).reshape(NW, k_chunks, CH)

    fs_a = feat_src[:, :DH]
    fs_b = feat_src[:, DH:]
    zrow = jnp.zeros((CH, DH), jnp.float32)
    zdeg = jnp.zeros((CH, DW), jnp.float32)
    ones = jnp.ones((CH, DW), jnp.float32)

    sums, degs = _sc_segment_sum(fs_a, fs_b, src_t, dst_t, zrow, zdeg, ones,
                                 acc_rows, k_chunks)

    fd_pad = jnp.pad(feat_dst, ((0, acc_rows - n_dst), (0, 0)))
    wnt = W_neigh.T
    out = _tc_combine(sums, degs, fd_pad, W_self.T, wnt[:DH], wnt[DH:],
                      b_self.reshape(1, d), b_neigh.reshape(1, d),
                      acc_rows, 512)
    return out[:n_dst]


# speedup vs baseline: 2.1295x; 1.0462x over previous
import jax
import jax.numpy as jnp
from jax import lax
from jax.experimental import pallas as pl
from jax.experimental.pallas import tpu as pltpu
from jax.experimental.pallas import tpu_sc as plsc

NC = 2
NS = 16
NW = NC * NS
CH = 128
DW = 8
DH = 64


def _sc_segment_sum(fs_a, fs_b, src_t, dst_t, zrow, zhist, acc_rows, k_chunks):
    share = acc_rows // NS

    def body(fsa_hbm, fsb_hbm, src_hbm, dst_hbm, zrow_hbm, zhist_hbm,
             sums_hbm, degs_hbm,
             src_v, dst_v, rows_a, zrow_v, hist_v, gsem_a, acc):
        c = lax.axis_index("c")
        s = lax.axis_index("s")
        wid = c * NS + s
        ones16 = jnp.ones((16,), jnp.float32)

        pltpu.sync_copy(src_hbm.at[wid], src_v)
        pltpu.sync_copy(dst_hbm.at[wid], dst_v)
        pltpu.sync_copy(zrow_hbm, zrow_v)
        pltpu.sync_copy(zhist_hbm, hist_v)

        base = s * share

        def zero_share():
            for k in range(share // CH):
                pltpu.sync_copy(zrow_v, acc.at[pl.ds(base + k * CH, CH)])

        def accumulate(feat_hbm, with_hist):
            def chunk(j, carry):
                cp = pltpu.async_copy(feat_hbm.at[src_v.at[j]], rows_a, gsem_a)
                if with_hist:
                    for l in range(CH // 16):
                        idx = dst_v[j, pl.ds(l * 16, 16)]
                        plsc.addupdate_scatter(hist_v, [idx], ones16)
                cp.wait()
                pltpu.sync_copy(rows_a, acc.at[dst_v.at[j]], add=True)
                return carry

            lax.fori_loop(0, k_chunks, chunk, 0)

        def write_share(half):
            for k in range(share // CH):
                r = base + k * CH
                pltpu.sync_copy(acc.at[pl.ds(r, CH)], rows_a)
                pltpu.sync_copy(rows_a, sums_hbm.at[c, half, pl.ds(r, CH)])

        zero_share()
        plsc.subcore_barrier()
        accumulate(fsa_hbm, with_hist=True)
        pltpu.sync_copy(hist_v, degs_hbm.at[wid])
        plsc.subcore_barrier()
        write_share(0)
        zero_share()
        plsc.subcore_barrier()
        accumulate(fsb_hbm, with_hist=False)
        plsc.subcore_barrier()
        write_share(1)

    mesh = plsc.VectorSubcoreMesh(core_axis_name="c", subcore_axis_name="s",
                                  num_cores=NC, num_subcores=NS)
    fn = pl.kernel(
        body,
        out_type=(jax.ShapeDtypeStruct((NC, 2, acc_rows, DH), jnp.float32),
                  jax.ShapeDtypeStruct((NW, acc_rows), jnp.float32)),
        mesh=mesh,
        compiler_params=pltpu.CompilerParams(use_tc_tiling_on_sc=False,
                                             needs_layout_passes=False),
        scratch_types=[
            pltpu.VMEM((k_chunks, CH), jnp.int32),
            pltpu.VMEM((k_chunks, CH), jnp.int32),
            pltpu.VMEM((CH, DH), jnp.float32),
            pltpu.VMEM((CH, DH), jnp.float32),
            pltpu.VMEM((acc_rows,), jnp.float32),
            pltpu.SemaphoreType.DMA,
            pltpu.VMEM_SHARED((acc_rows, DH), jnp.float32),
        ],
    )
    return fn(fs_a, fs_b, src_t, dst_t, zrow, zhist)


def _tc_combine(sums, degs, fd, wst, wnt_a, wnt_b, bs, bn, rows, blk):
    d = fd.shape[1]
    ones32 = jnp.ones((NW, 1), jnp.float32)

    def body(p_ref, d_ref, o32_ref, fd_ref, wst_ref, wa_ref, wb_ref, bs_ref,
             bn_ref, o_ref):
        sa = p_ref[0, 0] + p_ref[1, 0]
        sb = p_ref[0, 1] + p_ref[1, 1]
        deg = lax.dot_general(d_ref[...], o32_ref[...],
                              (((0,), (0,)), ((), ())),
                              preferred_element_type=jnp.float32)
        inv = 1.0 / jnp.maximum(deg, 1.0)
        o_ref[...] = (
            jnp.dot(fd_ref[...], wst_ref[...], preferred_element_type=jnp.float32)
            + jnp.dot(sa * inv, wa_ref[...], preferred_element_type=jnp.float32)
            + jnp.dot(sb * inv, wb_ref[...], preferred_element_type=jnp.float32)
            + bs_ref[...] + bn_ref[...]
        )

    return pl.pallas_call(
        body,
        grid=(rows // blk,),
        in_specs=[
            pl.BlockSpec((2, 2, blk, DH), lambda i: (0, 0, i, 0)),
            pl.BlockSpec((NW, blk), lambda i: (0, i)),
            pl.BlockSpec((NW, 1), lambda i: (0, 0)),
            pl.BlockSpec((blk, d), lambda i: (i, 0)),
            pl.BlockSpec((d, d), lambda i: (0, 0)),
            pl.BlockSpec((DH, d), lambda i: (0, 0)),
            pl.BlockSpec((DH, d), lambda i: (0, 0)),
            pl.BlockSpec((1, d), lambda i: (0, 0)),
            pl.BlockSpec((1, d), lambda i: (0, 0)),
        ],
        out_specs=pl.BlockSpec((blk, d), lambda i: (i, 0)),
        out_shape=jax.ShapeDtypeStruct((rows, d), jnp.float32),
    )(sums, degs, ones32, fd, wst, wnt_a, wnt_b, bs, bn)


def kernel(feat_src, feat_dst, edge_index, W_self, b_self, W_neigh, b_neigh):
    n_src, d = feat_src.shape
    n_dst = feat_dst.shape[0]
    e = edge_index.shape[1]

    k_chunks = -(-e // (NW * CH))
    e_pad = NW * k_chunks * CH
    acc_rows = -(-(n_dst + 1) // (NS * CH)) * (NS * CH)

    src = edge_index[0].astype(jnp.int32)
    dst = edge_index[1].astype(jnp.int32)
    pad = e_pad - e
    pad_cycle = jnp.arange(pad, dtype=jnp.int32) % CH
    src_t = jnp.concatenate([src, pad_cycle]).reshape(NW, k_chunks, CH)
    dst_t = jnp.concatenate([dst, n_dst + pad_cycle]).reshape(NW, k_chunks, CH)

    fs_a = feat_src[:, :DH]
    fs_b = feat_src[:, DH:]
    zrow = jnp.zeros((CH, DH), jnp.float32)
    zhist = jnp.zeros((acc_rows,), jnp.float32)

    sums, degs = _sc_segment_sum(fs_a, fs_b, src_t, dst_t, zrow, zhist,
                                 acc_rows, k_chunks)

    fd_pad = jnp.pad(feat_dst, ((0, acc_rows - n_dst), (0, 0)))
    wnt = W_neigh.T
    out = _tc_combine(sums, degs, fd_pad, W_self.T, wnt[:DH], wnt[DH:],
                      b_self.reshape(1, d), b_neigh.reshape(1, d),
                      acc_rows, 512)
    return out[:n_dst]


# speedup vs baseline: 2.1831x; 1.0252x over previous
import jax
import jax.numpy as jnp
from jax import lax
from jax.experimental import pallas as pl
from jax.experimental.pallas import tpu as pltpu
from jax.experimental.pallas import tpu_sc as plsc

NC = 2
NS = 16
NW = NC * NS
CH = 128
DW = 8
DH = 64


def _sc_segment_sum(fs_a, fs_b, src_t, dst_t, zrow, zhist, acc_rows, k_chunks):
    share = acc_rows // NS

    def body(fsa_hbm, fsb_hbm, src_hbm, dst_hbm, zrow_hbm, zhist_hbm,
             sums_hbm, degs_hbm,
             src_v, dst_v, rows_a, zrow_v, hist_v, gsem_a, acc):
        c = lax.axis_index("c")
        s = lax.axis_index("s")
        wid = c * NS + s
        ones16 = jnp.ones((16,), jnp.float32)

        pltpu.sync_copy(src_hbm.at[wid], src_v)
        pltpu.sync_copy(dst_hbm.at[wid], dst_v)
        pltpu.sync_copy(zrow_hbm, zrow_v)
        pltpu.sync_copy(zhist_hbm, hist_v)

        base = s * share

        def zero_share():
            for k in range(share // CH):
                pltpu.sync_copy(zrow_v, acc.at[pl.ds(base + k * CH, CH)])

        def accumulate(feat_hbm, with_hist):
            def chunk(j, carry):
                cp = pltpu.async_copy(feat_hbm.at[src_v.at[j]], rows_a, gsem_a)
                if with_hist:
                    for l in range(CH // 16):
                        idx = dst_v[j, pl.ds(l * 16, 16)]
                        plsc.addupdate_scatter(hist_v, [idx], ones16)
                cp.wait()
                pltpu.sync_copy(rows_a, acc.at[dst_v.at[j]], add=True)
                return carry

            lax.fori_loop(0, k_chunks, chunk, 0)

        def write_share(half):
            pltpu.sync_copy(acc.at[pl.ds(base, share)],
                            sums_hbm.at[c, half, pl.ds(base, share)])

        zero_share()
        plsc.subcore_barrier()
        accumulate(fsa_hbm, with_hist=True)
        pltpu.sync_copy(hist_v, degs_hbm.at[wid])
        plsc.subcore_barrier()
        write_share(0)
        zero_share()
        plsc.subcore_barrier()
        accumulate(fsb_hbm, with_hist=False)
        plsc.subcore_barrier()
        write_share(1)

    mesh = plsc.VectorSubcoreMesh(core_axis_name="c", subcore_axis_name="s",
                                  num_cores=NC, num_subcores=NS)
    fn = pl.kernel(
        body,
        out_type=(jax.ShapeDtypeStruct((NC, 2, acc_rows, DH), jnp.float32),
                  jax.ShapeDtypeStruct((NW, acc_rows), jnp.float32)),
        mesh=mesh,
        compiler_params=pltpu.CompilerParams(use_tc_tiling_on_sc=False,
                                             needs_layout_passes=False),
        scratch_types=[
            pltpu.VMEM((k_chunks, CH), jnp.int32),
            pltpu.VMEM((k_chunks, CH), jnp.int32),
            pltpu.VMEM((CH, DH), jnp.float32),
            pltpu.VMEM((CH, DH), jnp.float32),
            pltpu.VMEM((acc_rows,), jnp.float32),
            pltpu.SemaphoreType.DMA,
            pltpu.VMEM_SHARED((acc_rows, DH), jnp.float32),
        ],
    )
    return fn(fs_a, fs_b, src_t, dst_t, zrow, zhist)


def _tc_combine(sums, degs, fd, wst, wnt_a, wnt_b, bs, bn, rows, blk):
    d = fd.shape[1]
    ones32 = jnp.ones((NW, 1), jnp.float32)

    def body(p_ref, d_ref, o32_ref, fd_ref, wst_ref, wa_ref, wb_ref, bs_ref,
             bn_ref, o_ref):
        sa = p_ref[0, 0] + p_ref[1, 0]
        sb = p_ref[0, 1] + p_ref[1, 1]
        deg = jnp.dot(d_ref[...], o32_ref[...],
                      preferred_element_type=jnp.float32)
        inv = 1.0 / jnp.maximum(deg, 1.0)
        o_ref[...] = (
            jnp.dot(fd_ref[...], wst_ref[...], preferred_element_type=jnp.float32)
            + jnp.dot(sa * inv, wa_ref[...], preferred_element_type=jnp.float32)
            + jnp.dot(sb * inv, wb_ref[...], preferred_element_type=jnp.float32)
            + bs_ref[...] + bn_ref[...]
        )

    return pl.pallas_call(
        body,
        grid=(rows // blk,),
        in_specs=[
            pl.BlockSpec((2, 2, blk, DH), lambda i: (0, 0, i, 0)),
            pl.BlockSpec((blk, NW), lambda i: (i, 0)),
            pl.BlockSpec((NW, 1), lambda i: (0, 0)),
            pl.BlockSpec((blk, d), lambda i: (i, 0)),
            pl.BlockSpec((d, d), lambda i: (0, 0)),
            pl.BlockSpec((DH, d), lambda i: (0, 0)),
            pl.BlockSpec((DH, d), lambda i: (0, 0)),
            pl.BlockSpec((1, d), lambda i: (0, 0)),
            pl.BlockSpec((1, d), lambda i: (0, 0)),
        ],
        out_specs=pl.BlockSpec((blk, d), lambda i: (i, 0)),
        out_shape=jax.ShapeDtypeStruct((rows, d), jnp.float32),
    )(sums, degs, ones32, fd, wst, wnt_a, wnt_b, bs, bn)


def kernel(feat_src, feat_dst, edge_index, W_self, b_self, W_neigh, b_neigh):
    n_src, d = feat_src.shape
    n_dst = feat_dst.shape[0]
    e = edge_index.shape[1]

    k_chunks = -(-e // (NW * CH))
    e_pad = NW * k_chunks * CH
    acc_rows = -(-(n_dst + 1) // (NS * CH)) * (NS * CH)

    src = edge_index[0].astype(jnp.int32)
    dst = edge_index[1].astype(jnp.int32)
    pad = e_pad - e
    pad_cycle = jnp.arange(pad, dtype=jnp.int32) % CH
    src_t = jnp.concatenate([src, pad_cycle]).reshape(NW, k_chunks, CH)
    dst_t = jnp.concatenate([dst, n_dst + pad_cycle]).reshape(NW, k_chunks, CH)

    fs_a = feat_src[:, :DH]
    fs_b = feat_src[:, DH:]
    zrow = jnp.zeros((CH, DH), jnp.float32)
    zhist = jnp.zeros((acc_rows,), jnp.float32)

    sums, degs = _sc_segment_sum(fs_a, fs_b, src_t, dst_t, zrow, zhist,
                                 acc_rows, k_chunks)

    wnt = W_neigh.T
    return _tc_combine(sums, degs.T, feat_dst, W_self.T, wnt[:DH], wnt[DH:],
                       b_self.reshape(1, d), b_neigh.reshape(1, d),
                       n_dst, 1000)


# speedup vs baseline: 2.3084x; 1.0574x over previous
import jax
import jax.numpy as jnp
from jax import lax
from jax.experimental import pallas as pl
from jax.experimental.pallas import tpu as pltpu
from jax.experimental.pallas import tpu_sc as plsc

NC = 2
NS = 16
NW = NC * NS
CH = 128
DW = 8
DH = 64


def _sc_segment_sum(fs_a, fs_b, src_t, dst_t, zrow, zhist, acc_rows, k_chunks):
    share = acc_rows // NS

    def body(fsa_hbm, fsb_hbm, src_hbm, dst_hbm, zrow_hbm, zhist_hbm,
             sums_hbm, degs_hbm,
             src_v, dst_v, rows_a, zrow_v, hist_v, gsem_a, acc):
        c = lax.axis_index("c")
        s = lax.axis_index("s")
        ones16 = jnp.ones((16,), jnp.float32)

        pltpu.sync_copy(src_hbm.at[s], src_v)
        pltpu.sync_copy(dst_hbm.at[s], dst_v)
        pltpu.sync_copy(zrow_hbm, zrow_v)
        pltpu.sync_copy(zhist_hbm, hist_v)

        base = s * share
        for k in range(share // CH):
            pltpu.sync_copy(zrow_v, acc.at[pl.ds(base + k * CH, CH)])
        plsc.subcore_barrier()

        def accumulate(feat_hbm, with_hist):
            def chunk(j, carry):
                cp = pltpu.async_copy(feat_hbm.at[src_v.at[j]], rows_a, gsem_a)
                if with_hist:
                    for l in range(CH // 16):
                        idx = dst_v[j, pl.ds(l * 16, 16)]
                        plsc.addupdate_scatter(hist_v, [idx], ones16)
                cp.wait()
                pltpu.sync_copy(rows_a, acc.at[dst_v.at[j]], add=True)
                return carry

            lax.fori_loop(0, k_chunks, chunk, 0)

        @pl.when(c == 0)
        def _():
            accumulate(fsa_hbm, with_hist=True)
            pltpu.sync_copy(hist_v, degs_hbm.at[s])

        @pl.when(c == 1)
        def _():
            accumulate(fsb_hbm, with_hist=False)

        plsc.subcore_barrier()
        pltpu.sync_copy(acc.at[pl.ds(base, share)],
                        sums_hbm.at[c, pl.ds(base, share)])

    mesh = plsc.VectorSubcoreMesh(core_axis_name="c", subcore_axis_name="s",
                                  num_cores=NC, num_subcores=NS)
    fn = pl.kernel(
        body,
        out_type=(jax.ShapeDtypeStruct((NC, acc_rows, DH), jnp.float32),
                  jax.ShapeDtypeStruct((NS, acc_rows), jnp.float32)),
        mesh=mesh,
        compiler_params=pltpu.CompilerParams(use_tc_tiling_on_sc=False,
                                             needs_layout_passes=False),
        scratch_types=[
            pltpu.VMEM((k_chunks, CH), jnp.int32),
            pltpu.VMEM((k_chunks, CH), jnp.int32),
            pltpu.VMEM((CH, DH), jnp.float32),
            pltpu.VMEM((CH, DH), jnp.float32),
            pltpu.VMEM((acc_rows,), jnp.float32),
            pltpu.SemaphoreType.DMA,
            pltpu.VMEM_SHARED((acc_rows, DH), jnp.float32),
        ],
    )
    return fn(fs_a, fs_b, src_t, dst_t, zrow, zhist)


def _tc_combine(sums, degs, fd, wst, wnt_a, wnt_b, bs, bn, rows, blk):
    d = fd.shape[1]
    ones16 = jnp.ones((NS, 1), jnp.float32)

    def body(p_ref, d_ref, o16_ref, fd_ref, wst_ref, wa_ref, wb_ref, bs_ref,
             bn_ref, o_ref):
        sa = p_ref[0]
        sb = p_ref[1]
        deg = jnp.dot(d_ref[...], o16_ref[...],
                      preferred_element_type=jnp.float32)
        inv = 1.0 / jnp.maximum(deg, 1.0)
        o_ref[...] = (
            jnp.dot(fd_ref[...], wst_ref[...], preferred_element_type=jnp.float32)
            + jnp.dot(sa * inv, wa_ref[...], preferred_element_type=jnp.float32)
            + jnp.dot(sb * inv, wb_ref[...], preferred_element_type=jnp.float32)
            + bs_ref[...] + bn_ref[...]
        )

    return pl.pallas_call(
        body,
        grid=(rows // blk,),
        in_specs=[
            pl.BlockSpec((2, blk, DH), lambda i: (0, i, 0)),
            pl.BlockSpec((blk, NS), lambda i: (i, 0)),
            pl.BlockSpec((NS, 1), lambda i: (0, 0)),
            pl.BlockSpec((blk, d), lambda i: (i, 0)),
            pl.BlockSpec((d, d), lambda i: (0, 0)),
            pl.BlockSpec((DH, d), lambda i: (0, 0)),
            pl.BlockSpec((DH, d), lambda i: (0, 0)),
            pl.BlockSpec((1, d), lambda i: (0, 0)),
            pl.BlockSpec((1, d), lambda i: (0, 0)),
        ],
        out_specs=pl.BlockSpec((blk, d), lambda i: (i, 0)),
        out_shape=jax.ShapeDtypeStruct((rows, d), jnp.float32),
    )(sums, degs, ones16, fd, wst, wnt_a, wnt_b, bs, bn)


def kernel(feat_src, feat_dst, edge_index, W_self, b_self, W_neigh, b_neigh):
    n_src, d = feat_src.shape
    n_dst = feat_dst.shape[0]
    e = edge_index.shape[1]

    k_chunks = -(-e // (NS * CH))
    e_pad = NS * k_chunks * CH
    acc_rows = -(-(n_dst + 1) // (NS * CH)) * (NS * CH)

    src = edge_index[0].astype(jnp.int32)
    dst = edge_index[1].astype(jnp.int32)
    pad = e_pad - e
    pad_cycle = jnp.arange(pad, dtype=jnp.int32) % CH
    src_t = jnp.concatenate([src, pad_cycle]).reshape(NS, k_chunks, CH)
    dst_t = jnp.concatenate([dst, n_dst + pad_cycle]).reshape(NS, k_chunks, CH)

    fs_a = feat_src[:, :DH]
    fs_b = feat_src[:, DH:]
    zrow = jnp.zeros((CH, DH), jnp.float32)
    zhist = jnp.zeros((acc_rows,), jnp.float32)

    sums, degs = _sc_segment_sum(fs_a, fs_b, src_t, dst_t, zrow, zhist,
                                 acc_rows, k_chunks)

    wnt = W_neigh.T
    return _tc_combine(sums, degs.T, feat_dst, W_self.T, wnt[:DH], wnt[DH:],
                       b_self.reshape(1, d), b_neigh.reshape(1, d),
                       n_dst, 1000)


# speedup vs baseline: 2.3198x; 1.0049x over previous
import jax
import jax.numpy as jnp
from jax import lax
from jax.experimental import pallas as pl
from jax.experimental.pallas import tpu as pltpu
from jax.experimental.pallas import tpu_sc as plsc

NC = 2
NS = 16
NW = NC * NS
CH = 128
DW = 8
DH = 64


def _sc_segment_sum(fs_a, fs_b, src_t, dst_t, zrow, zhist, acc_rows, k_chunks):
    share = acc_rows // NS

    def body(fsa_hbm, fsb_hbm, src_hbm, dst_hbm, zrow_hbm, zhist_hbm,
             sums_hbm, degs_hbm,
             src_v, dst_v, rows_a, hist_v, gsem_a, acc):
        c = lax.axis_index("c")
        s = lax.axis_index("s")
        ones16 = jnp.ones((16,), jnp.float32)
        base = s * share

        cp1 = pltpu.async_copy(src_hbm.at[s], src_v, gsem_a)
        cp2 = pltpu.async_copy(dst_hbm.at[s], dst_v, gsem_a)
        cp3 = pltpu.async_copy(zhist_hbm, hist_v, gsem_a)
        pltpu.sync_copy(zrow_hbm, acc.at[pl.ds(base, share)])
        cp3.wait()
        cp2.wait()
        cp1.wait()
        plsc.subcore_barrier()

        def accumulate(feat_hbm, with_hist):
            def chunk(j, carry):
                cp = pltpu.async_copy(feat_hbm.at[src_v.at[j]], rows_a, gsem_a)
                if with_hist:
                    for l in range(CH // 16):
                        idx = dst_v[j, pl.ds(l * 16, 16)]
                        plsc.addupdate_scatter(hist_v, [idx], ones16)
                cp.wait()
                pltpu.sync_copy(rows_a, acc.at[dst_v.at[j]], add=True)
                return carry

            lax.fori_loop(0, k_chunks, chunk, 0)

        @pl.when(c == 0)
        def _():
            accumulate(fsa_hbm, with_hist=True)
            pltpu.sync_copy(hist_v, degs_hbm.at[s])

        @pl.when(c == 1)
        def _():
            accumulate(fsb_hbm, with_hist=False)

        plsc.subcore_barrier()
        pltpu.sync_copy(acc.at[pl.ds(base, share)],
                        sums_hbm.at[c, pl.ds(base, share)])

    mesh = plsc.VectorSubcoreMesh(core_axis_name="c", subcore_axis_name="s",
                                  num_cores=NC, num_subcores=NS)
    fn = pl.kernel(
        body,
        out_type=(jax.ShapeDtypeStruct((NC, acc_rows, DH), jnp.float32),
                  jax.ShapeDtypeStruct((NS, acc_rows), jnp.float32)),
        mesh=mesh,
        compiler_params=pltpu.CompilerParams(use_tc_tiling_on_sc=False,
                                             needs_layout_passes=False),
        scratch_types=[
            pltpu.VMEM((k_chunks, CH), jnp.int32),
            pltpu.VMEM((k_chunks, CH), jnp.int32),
            pltpu.VMEM((CH, DH), jnp.float32),
            pltpu.VMEM((acc_rows,), jnp.float32),
            pltpu.SemaphoreType.DMA,
            pltpu.VMEM_SHARED((acc_rows, DH), jnp.float32),
        ],
    )
    return fn(fs_a, fs_b, src_t, dst_t, zrow, zhist)


def _tc_combine(sums, degs, fd, wst, wnt_a, wnt_b, bs, bn, rows, blk):
    d = fd.shape[1]
    ones16 = jnp.ones((NS, 1), jnp.float32)

    def body(p_ref, d_ref, o16_ref, fd_ref, wst_ref, wa_ref, wb_ref, bs_ref,
             bn_ref, o_ref):
        sa = p_ref[0]
        sb = p_ref[1]
        deg = jnp.dot(d_ref[...], o16_ref[...],
                      preferred_element_type=jnp.float32)
        inv = 1.0 / jnp.maximum(deg, 1.0)
        o_ref[...] = (
            jnp.dot(fd_ref[...], wst_ref[...], preferred_element_type=jnp.float32)
            + jnp.dot(sa * inv, wa_ref[...], preferred_element_type=jnp.float32)
            + jnp.dot(sb * inv, wb_ref[...], preferred_element_type=jnp.float32)
            + bs_ref[...] + bn_ref[...]
        )

    return pl.pallas_call(
        body,
        grid=(rows // blk,),
        in_specs=[
            pl.BlockSpec((2, blk, DH), lambda i: (0, i, 0)),
            pl.BlockSpec((blk, NS), lambda i: (i, 0)),
            pl.BlockSpec((NS, 1), lambda i: (0, 0)),
            pl.BlockSpec((blk, d), lambda i: (i, 0)),
            pl.BlockSpec((d, d), lambda i: (0, 0)),
            pl.BlockSpec((DH, d), lambda i: (0, 0)),
            pl.BlockSpec((DH, d), lambda i: (0, 0)),
            pl.BlockSpec((1, d), lambda i: (0, 0)),
            pl.BlockSpec((1, d), lambda i: (0, 0)),
        ],
        out_specs=pl.BlockSpec((blk, d), lambda i: (i, 0)),
        out_shape=jax.ShapeDtypeStruct((rows, d), jnp.float32),
    )(sums, degs, ones16, fd, wst, wnt_a, wnt_b, bs, bn)


def kernel(feat_src, feat_dst, edge_index, W_self, b_self, W_neigh, b_neigh):
    n_src, d = feat_src.shape
    n_dst = feat_dst.shape[0]
    e = edge_index.shape[1]

    k_chunks = -(-e // (NS * CH))
    e_pad = NS * k_chunks * CH
    acc_rows = -(-(n_dst + 1) // (NS * CH)) * (NS * CH)

    src = edge_index[0].astype(jnp.int32)
    dst = edge_index[1].astype(jnp.int32)
    pad = e_pad - e
    pad_cycle = jnp.arange(pad, dtype=jnp.int32) % CH
    src_t = jnp.concatenate([src, pad_cycle]).reshape(NS, k_chunks, CH)
    dst_t = jnp.concatenate([dst, n_dst + pad_cycle]).reshape(NS, k_chunks, CH)

    fs_a = feat_src[:, :DH]
    fs_b = feat_src[:, DH:]
    zrow = jnp.zeros((acc_rows // NS, DH), jnp.float32)
    zhist = jnp.zeros((acc_rows,), jnp.float32)

    sums, degs = _sc_segment_sum(fs_a, fs_b, src_t, dst_t, zrow, zhist,
                                 acc_rows, k_chunks)

    wnt = W_neigh.T
    return _tc_combine(sums, degs.T, feat_dst, W_self.T, wnt[:DH], wnt[DH:],
                       b_self.reshape(1, d), b_neigh.reshape(1, d),
                       n_dst, 1000)
